# baseline (speedup 1.0000x reference)
import functools

import jax
import jax.numpy as jnp
from jax import lax
from jax.experimental import pallas as pl
from jax.experimental.pallas import tpu as pltpu
from jax.experimental.pallas import tpu_sc as plsc

_B, _L = 4096, 200
_LOC_DIM, _USER_DIM, _HID = 64, 16, 80
_N = _B * _L

_NW = 32
_PER_W = _N // _NW
_C = 512
_NCHUNK = _PER_W // _C
_LCH = 4


def _sc_gather(loc_table, user_table, lids, uids):
    mesh = plsc.VectorSubcoreMesh(core_axis_name="core", subcore_axis_name="subcore")

    @functools.partial(
        pl.kernel,
        out_type=jax.ShapeDtypeStruct((_N, 128), jnp.float32),
        mesh=mesh,
        scratch_types=[
            pltpu.VMEM((_C,), jnp.int32),
            pltpu.VMEM((_C,), jnp.int32),
            pltpu.VMEM((_C, 128), jnp.float32),
            pltpu.VMEM((_C, _USER_DIM), jnp.float32),
            pltpu.SemaphoreType.DMA,
            pltpu.SemaphoreType.DMA,
        ],
        compiler_params=pltpu.CompilerParams(use_tc_tiling_on_sc=False),
    )
    def gather_kernel(loc_hbm, user_hbm, lid_hbm, uid_hbm, out_hbm,
                      li_v, ui_v, lrows_v, urows_v, sem_l, sem_u):
        wid = lax.axis_index("subcore") * 2 + lax.axis_index("core")
        base = wid * _PER_W

        @pl.loop(0, _NCHUNK)
        def _(c):
            row = base + c * _C
            pltpu.sync_copy(lid_hbm.at[pl.ds(row, _C)], li_v)
            pltpu.sync_copy(uid_hbm.at[pl.ds(row, _C)], ui_v)
            cl = pltpu.async_copy(loc_hbm.at[li_v], lrows_v, sem_l)
            cu = pltpu.async_copy(user_hbm.at[ui_v], urows_v, sem_u)
            cl.wait()
            cu.wait()
            pltpu.sync_copy(lrows_v, out_hbm.at[pl.ds(row, _C)])
            pltpu.sync_copy(urows_v,
                            out_hbm.at[pl.ds(row, _C), pl.ds(_LOC_DIM, _USER_DIM)])

    return gather_kernel(loc_table, user_table, lids, uids)


def _mlp_body(emb_ref, w_ref, b_ref, g_ref, bt_ref, out_ref):
    for l in range(_LCH):
        e = emb_ref[pl.ds(l * _B, _B), :]
        hT = jax.lax.dot_general(
            w_ref[...], e,
            dimension_numbers=(((1,), (1,)), ((), ())),
            preferred_element_type=jnp.float32)
        hT += b_ref[...]
        mu = jnp.mean(hT, axis=0, keepdims=True)
        var = jnp.mean((hT - mu) ** 2, axis=0, keepdims=True)
        y = (hT - mu) * jax.lax.rsqrt(var + 1e-5) * g_ref[...] + bt_ref[...]
        out_ref[l] = 0.5 * y * (1.0 + jax.lax.erf(y * 0.7071067811865476))


def _mlp(emb, w128, b, gamma, beta):
    out3 = pl.pallas_call(
        _mlp_body,
        grid=(_L // _LCH,),
        in_specs=[
            pl.BlockSpec((_LCH * _B, 128), lambda i: (i, 0)),
            pl.BlockSpec((_HID, 128), lambda i: (0, 0)),
            pl.BlockSpec((_HID, 1), lambda i: (0, 0)),
            pl.BlockSpec((_HID, 1), lambda i: (0, 0)),
            pl.BlockSpec((_HID, 1), lambda i: (0, 0)),
        ],
        out_specs=pl.BlockSpec((_LCH, _HID, _B), lambda i: (i, 0, 0)),
        out_shape=jax.ShapeDtypeStruct((_L, _HID, _B), jnp.float32),
        compiler_params=pltpu.CompilerParams(
            dimension_semantics=("arbitrary",)),
    )(emb, w128, b, gamma, beta)
    return out3.transpose(2, 0, 1)


def kernel(loc_ids, user_ids, loc_table, user_table, W, b, gamma, beta):
    lids = loc_ids.T.reshape(_N)
    uids = user_ids.T.reshape(_N)
    loc_pad = jnp.pad(loc_table, ((0, 0), (0, 128 - _LOC_DIM)))
    emb = _sc_gather(loc_pad, user_table, lids, uids)
    w128 = jnp.pad(W, ((0, 0), (0, 128 - _HID)))
    return _mlp(emb, w128,
                b.reshape(_HID, 1), gamma.reshape(_HID, 1),
                beta.reshape(_HID, 1))


# speedup vs baseline: 1.1223x; 1.1223x over previous
import functools

import jax
import jax.numpy as jnp
from jax import lax
from jax.experimental import pallas as pl
from jax.experimental.pallas import tpu as pltpu
from jax.experimental.pallas import tpu_sc as plsc

_B, _L = 4096, 200
_LOC_DIM, _USER_DIM, _HID = 64, 16, 80
_N = _B * _L

_NSPLIT = 4
_LSPL = _L // _NSPLIT
_NTOK = _B * _LSPL

_NW = 32
_PER_W = _NTOK // _NW
_C = 640
_NCHUNK = _PER_W // _C
_LCH = 4


def _sc_gather(loc_table, user_table, lids, uids):
    mesh = plsc.VectorSubcoreMesh(core_axis_name="core", subcore_axis_name="subcore")

    @functools.partial(
        pl.kernel,
        out_type=jax.ShapeDtypeStruct((_NTOK, 128), jnp.float32),
        mesh=mesh,
        scratch_types=[
            pltpu.VMEM((_C,), jnp.int32),
            pltpu.VMEM((_C,), jnp.int32),
            pltpu.VMEM((_C, _LOC_DIM), jnp.float32),
            pltpu.VMEM((_C, _USER_DIM), jnp.float32),
            pltpu.SemaphoreType.DMA,
            pltpu.SemaphoreType.DMA,
        ],
        compiler_params=pltpu.CompilerParams(use_tc_tiling_on_sc=False),
    )
    def gather_kernel(loc_hbm, user_hbm, lid_hbm, uid_hbm, out_hbm,
                      li_v, ui_v, lrows_v, urows_v, sem_l, sem_u):
        wid = lax.axis_index("subcore") * 2 + lax.axis_index("core")
        base = wid * _PER_W

        @pl.loop(0, _NCHUNK)
        def _(c):
            row = base + c * _C
            pltpu.sync_copy(lid_hbm.at[pl.ds(row, _C)], li_v)
            pltpu.sync_copy(uid_hbm.at[pl.ds(row, _C)], ui_v)
            cl = pltpu.async_copy(loc_hbm.at[li_v], lrows_v, sem_l)
            cu = pltpu.async_copy(user_hbm.at[ui_v], urows_v, sem_u)
            cl.wait()
            cu.wait()
            pltpu.sync_copy(lrows_v, out_hbm.at[pl.ds(row, _C), pl.ds(0, _LOC_DIM)])
            pltpu.sync_copy(urows_v,
                            out_hbm.at[pl.ds(row, _C), pl.ds(_LOC_DIM, _USER_DIM)])

    return gather_kernel(loc_table, user_table, lids, uids)


def _mlp_body(emb_ref, wl_ref, wu_ref, b_ref, g_ref, bt_ref, out_ref):
    for l in range(_LCH):
        e = emb_ref[pl.ds(l * _B, _B), :]
        hT = jax.lax.dot_general(
            wl_ref[...], e[:, :_LOC_DIM],
            dimension_numbers=(((1,), (1,)), ((), ())),
            preferred_element_type=jnp.float32)
        hT += jax.lax.dot_general(
            wu_ref[...], e[:, _LOC_DIM:_LOC_DIM + _USER_DIM],
            dimension_numbers=(((1,), (1,)), ((), ())),
            preferred_element_type=jnp.float32)
        hT += b_ref[...]
        mu = jnp.mean(hT, axis=0, keepdims=True)
        var = jnp.mean((hT - mu) ** 2, axis=0, keepdims=True)
        y = (hT - mu) * jax.lax.rsqrt(var + 1e-5) * g_ref[...] + bt_ref[...]
        out_ref[l] = 0.5 * y * (1.0 + jax.lax.erf(y * 0.7071067811865476))


def _mlp_chunk(emb, wl, wu, b, gamma, beta, prev, k):
    nsteps = _LSPL // _LCH
    in_specs = [
        pl.BlockSpec((_LCH * _B, 128), lambda i: (i, 0)),
        pl.BlockSpec((_HID, _LOC_DIM), lambda i: (0, 0)),
        pl.BlockSpec((_HID, _USER_DIM), lambda i: (0, 0)),
        pl.BlockSpec((_HID, 1), lambda i: (0, 0)),
        pl.BlockSpec((_HID, 1), lambda i: (0, 0)),
        pl.BlockSpec((_HID, 1), lambda i: (0, 0)),
    ]
    args = [emb, wl, wu, b, gamma, beta]
    io_aliases = {}
    body = _mlp_body
    if prev is not None:
        in_specs.append(pl.BlockSpec(memory_space=pl.ANY))
        args.append(prev)
        io_aliases = {6: 0}

        def body(emb_ref, wl_ref, wu_ref, b_ref, g_ref, bt_ref, prev_ref,
                 out_ref):
            del prev_ref
            _mlp_body(emb_ref, wl_ref, wu_ref, b_ref, g_ref, bt_ref, out_ref)

    return pl.pallas_call(
        body,
        grid=(nsteps,),
        in_specs=in_specs,
        out_specs=pl.BlockSpec((_LCH, _HID, _B),
                               lambda i, k=k, n=nsteps: (k * n + i, 0, 0)),
        out_shape=jax.ShapeDtypeStruct((_L, _HID, _B), jnp.float32),
        input_output_aliases=io_aliases,
        compiler_params=pltpu.CompilerParams(
            dimension_semantics=("arbitrary",)),
    )(*args)


def kernel(loc_ids, user_ids, loc_table, user_table, W, b, gamma, beta):
    lids = loc_ids.T.reshape(_N)
    uids = user_ids.T.reshape(_N)
    wl = W[:, :_LOC_DIM]
    wu = W[:, _LOC_DIM:]
    bc = b.reshape(_HID, 1)
    gc = gamma.reshape(_HID, 1)
    btc = beta.reshape(_HID, 1)
    out3 = None
    for k in range(_NSPLIT):
        lo = k * _NTOK
        emb = _sc_gather(loc_table, user_table,
                         lax.dynamic_slice(lids, (lo,), (_NTOK,)),
                         lax.dynamic_slice(uids, (lo,), (_NTOK,)))
        out3 = _mlp_chunk(emb, wl, wu, bc, gc, btc, out3, k)
    return out3.transpose(2, 0, 1)
